# Initial kernel scaffold; baseline (speedup 1.0000x reference)
import jax
import jax.numpy as jnp
from jax.experimental import pallas as pl


def kernel(label):
    raise NotImplementedError("write your pallas kernel here")


# speedup vs baseline: 49.0288x; 49.0288x over previous
import jax
import jax.numpy as jnp
from jax import lax
from jax.experimental import pallas as pl
from jax.experimental.pallas import tpu as pltpu
from jax.experimental.pallas import tpu_sc as plsc

_CLASS_IDS = (3, 4, 5, 6, 7, 11, 16, 25, 32, 35)
_K = len(_CLASS_IDS)
_H = _W = 1024
_N = _H * _W
_NC, _NS, _L = 2, 16, 16
_NW = _NC * _NS
_PER_W = _N // _NW
_C = 8192
_CHUNKS = _PER_W // _C


def _onehot_body(lab_hbm, out_hbm, lab_v, out_v, sem):
    wid = lax.axis_index("s") * _NC + lax.axis_index("c")
    base = wid * _PER_W
    ones = jnp.full((_L,), 1.0, jnp.float32)
    zeros = jnp.zeros((_L,), jnp.float32)

    for t in range(_CHUNKS):
        start = base + t * _C
        pltpu.sync_copy(lab_hbm.at[pl.ds(start, _C)], lab_v)

        def g_body(g, carry):
            v = lab_v[pl.ds(g * _L, _L)]
            for k, cid in enumerate(_CLASS_IDS):
                out_v[pl.ds(k * _C + g * _L, _L)] = jnp.where(v == cid, ones, zeros)
            return carry

        lax.fori_loop(0, _C // _L, g_body, 0)

        descs = [
            pltpu.async_copy(
                out_v.at[pl.ds(k * _C, _C)],
                out_hbm.at[pl.ds(k * _N + start, _C)],
                sem,
            )
            for k in range(_K)
        ]
        for d in descs:
            d.wait()


def kernel(label):
    lab = label.reshape(_N).astype(jnp.int32)
    out = pl.kernel(
        _onehot_body,
        out_type=jax.ShapeDtypeStruct((_K * _N,), jnp.float32),
        mesh=plsc.VectorSubcoreMesh(
            core_axis_name="c", subcore_axis_name="s",
            num_cores=_NC, num_subcores=_NS,
        ),
        scratch_types=[
            pltpu.VMEM((_C,), jnp.int32),
            pltpu.VMEM((_K * _C,), jnp.float32),
            pltpu.SemaphoreType.DMA,
        ],
    )(lab)
    return out.reshape(_K, _H, _W)


# speedup vs baseline: 56.6034x; 1.1545x over previous
import jax
import jax.numpy as jnp
from jax import lax
from jax.experimental import pallas as pl
from jax.experimental.pallas import tpu as pltpu
from jax.experimental.pallas import tpu_sc as plsc

_CLASS_IDS = (3, 4, 5, 6, 7, 11, 16, 25, 32, 35)
_K = len(_CLASS_IDS)
_H = _W = 1024
_N = _H * _W
_NC, _NS, _L = 2, 16, 16
_NW = _NC * _NS
_PER_W = _N // _NW
_C = 4096
_CHUNKS = _PER_W // _C


def _onehot_body(lab_hbm, out_hbm,
                 lab0, lab1, out0, out1,
                 in_sem0, in_sem1, out_sem0, out_sem1):
    wid = lax.axis_index("s") * _NC + lax.axis_index("c")
    base = wid * _PER_W
    ones = jnp.full((_L,), 1.0, jnp.float32)
    zeros = jnp.zeros((_L,), jnp.float32)
    labs = (lab0, lab1)
    outs = (out0, out1)
    in_sems = (in_sem0, in_sem1)
    out_sems = (out_sem0, out_sem1)

    def fetch(t):
        return pltpu.async_copy(
            lab_hbm.at[pl.ds(base + t * _C, _C)], labs[t % 2], in_sems[t % 2])

    in_descs = {0: fetch(0)}
    out_descs = {}

    for t in range(_CHUNKS):
        b = t % 2
        if t + 1 < _CHUNKS:
            in_descs[t + 1] = fetch(t + 1)
        in_descs[t].wait()
        if t >= 2:
            for d in out_descs[t - 2]:
                d.wait()

        lab_v, out_v = labs[b], outs[b]

        def g_body(g, carry):
            v = lab_v[pl.ds(g * _L, _L)]
            for k, cid in enumerate(_CLASS_IDS):
                out_v[pl.ds(k * _C + g * _L, _L)] = jnp.where(v == cid, ones, zeros)
            return carry

        lax.fori_loop(0, _C // _L, g_body, 0)

        out_descs[t] = [
            pltpu.async_copy(
                out_v.at[pl.ds(k * _C, _C)],
                out_hbm.at[pl.ds(k * _N + base + t * _C, _C)],
                out_sems[b],
            )
            for k in range(_K)
        ]

    for t in (_CHUNKS - 2, _CHUNKS - 1):
        for d in out_descs[t]:
            d.wait()


def kernel(label):
    lab = label.reshape(_N).astype(jnp.int32)
    out = pl.kernel(
        _onehot_body,
        out_type=jax.ShapeDtypeStruct((_K * _N,), jnp.float32),
        mesh=plsc.VectorSubcoreMesh(
            core_axis_name="c", subcore_axis_name="s",
            num_cores=_NC, num_subcores=_NS,
        ),
        scratch_types=[
            pltpu.VMEM((_C,), jnp.int32),
            pltpu.VMEM((_C,), jnp.int32),
            pltpu.VMEM((_K * _C,), jnp.float32),
            pltpu.VMEM((_K * _C,), jnp.float32),
            pltpu.SemaphoreType.DMA,
            pltpu.SemaphoreType.DMA,
            pltpu.SemaphoreType.DMA,
            pltpu.SemaphoreType.DMA,
        ],
    )(lab)
    return out.reshape(_K, _H, _W)


# speedup vs baseline: 129.5079x; 2.2880x over previous
import jax
import jax.numpy as jnp
from jax import lax
from jax.experimental import pallas as pl
from jax.experimental.pallas import tpu as pltpu
from jax.experimental.pallas import tpu_sc as plsc

_CLASS_IDS = (3, 4, 5, 6, 7, 11, 16, 25, 32, 35)
_K = len(_CLASS_IDS)
_H = _W = 1024
_NC, _NS, _L = 2, 16, 16
_NW = _NC * _NS
_ROWS_W = _H // _NW
_R = 4
_CHUNKS = _ROWS_W // _R
_GROUPS = _R * _W // _L


def _onehot_body(lab_hbm, out_hbm,
                 lab0, lab1, out0, out1,
                 in_sem0, in_sem1, out_sem0, out_sem1):
    wid = lax.axis_index("s") * _NC + lax.axis_index("c")
    row0 = wid * _ROWS_W
    ones = jnp.full((_L,), 1.0, jnp.float32)
    zeros = jnp.zeros((_L,), jnp.float32)
    labs = (lab0, lab1)
    outs = (out0, out1)
    in_sems = (in_sem0, in_sem1)
    out_sems = (out_sem0, out_sem1)

    def fetch(t):
        return pltpu.async_copy(
            lab_hbm.at[pl.ds(row0 + t * _R, _R), :], labs[t % 2], in_sems[t % 2])

    in_descs = {0: fetch(0)}
    out_descs = {}

    for t in range(_CHUNKS):
        b = t % 2
        if t + 1 < _CHUNKS:
            in_descs[t + 1] = fetch(t + 1)
        in_descs[t].wait()
        if t >= 2:
            for d in out_descs[t - 2]:
                d.wait()

        lab_v, out_v = labs[b], outs[b]

        def g_body(g, carry):
            r = g // (_W // _L)
            c = (g % (_W // _L)) * _L
            v = lab_v[r, pl.ds(c, _L)]
            for k, cid in enumerate(_CLASS_IDS):
                out_v[k, r, pl.ds(c, _L)] = jnp.where(v == cid, ones, zeros)
            return carry

        lax.fori_loop(0, _GROUPS, g_body, 0)

        out_descs[t] = [
            pltpu.async_copy(
                out_v.at[k],
                out_hbm.at[k, pl.ds(row0 + t * _R, _R), :],
                out_sems[b],
            )
            for k in range(_K)
        ]

    for t in (_CHUNKS - 2, _CHUNKS - 1):
        for d in out_descs[t]:
            d.wait()


def kernel(label):
    lab = label.astype(jnp.int32)
    return pl.kernel(
        _onehot_body,
        out_type=jax.ShapeDtypeStruct((_K, _H, _W), jnp.float32),
        mesh=plsc.VectorSubcoreMesh(
            core_axis_name="c", subcore_axis_name="s",
            num_cores=_NC, num_subcores=_NS,
        ),
        scratch_types=[
            pltpu.VMEM((_R, _W), jnp.int32),
            pltpu.VMEM((_R, _W), jnp.int32),
            pltpu.VMEM((_K, _R, _W), jnp.float32),
            pltpu.VMEM((_K, _R, _W), jnp.float32),
            pltpu.SemaphoreType.DMA,
            pltpu.SemaphoreType.DMA,
            pltpu.SemaphoreType.DMA,
            pltpu.SemaphoreType.DMA,
        ],
    )(lab)
